# baseline (speedup 1.0000x reference)
import functools

import jax
import jax.numpy as jnp
from jax import lax
from jax.experimental import pallas as pl
from jax.experimental.pallas import tpu as pltpu
from jax.experimental.pallas import tpu_sc as plsc

K = 16
ROWS = 8192
COLS = 8192
L = 16
NC = 2
NS = 16
NW = NC * NS
RPW = ROWS // NW
NV = COLS // L
UNROLL = 8


def _sc_body(thr_hbm, y_hbm, s_hbm, o_hbm, buf, yqbuf, tpv, thrv, yv, sem):
    wid = lax.axis_index("s") * NC + lax.axis_index("c")
    base_row = wid * RPW

    pltpu.sync_copy(thr_hbm, thrv)
    pltpu.sync_copy(y_hbm, yv)
    thr = thrv[...]
    yvec = yv[...]

    def do_row(r, carry):
        row = base_row + r
        pltpu.sync_copy(s_hbm.at[row], buf)

        def p1(i, mx):
            b = i * (L * UNROLL)
            for j in range(UNROLL):
                mx = jnp.maximum(mx, buf[pl.ds(b + j * L, L)])
            return mx
        mx = lax.fori_loop(0, NV // UNROLL, p1,
                           jnp.full((L,), -jnp.inf, jnp.float32))
        m = jnp.max(mx)

        def p2(i, zacc):
            b = i * (L * UNROLL)
            for j in range(UNROLL):
                e = jnp.exp(buf[pl.ds(b + j * L, L)] - m)
                buf[pl.ds(b + j * L, L)] = e
                zacc = zacc + e
            return zacc
        zv = lax.fori_loop(0, NV // UNROLL, p2, jnp.zeros((L,), jnp.float32))
        z = jnp.sum(zv)

        tprow = thr * z
        tpv[...] = tprow
        t7 = tprow[7]

        def p3(i, dacc):
            b = i * (L * UNROLL)
            for j in range(UNROLL):
                e = buf[pl.ds(b + j * L, L)]
                gt = (e > t7).astype(jnp.int32)
                idx = gt * 8
                tv = plsc.load_gather(tpv, [idx + 3])
                idx = idx + (e > tv).astype(jnp.int32) * 4
                tv = plsc.load_gather(tpv, [idx + 1])
                idx = idx + (e > tv).astype(jnp.int32) * 2
                tv = plsc.load_gather(tpv, [idx])
                idx = idx + (e > tv).astype(jnp.int32)
                yq = plsc.load_gather(yv, [idx])
                yqbuf[pl.ds(b + j * L, L)] = yq
                dacc = dacc + yq
            return dacc
        dv = lax.fori_loop(0, NV // UNROLL, p3, jnp.zeros((L,), jnp.float32))
        denom = jnp.maximum(jnp.sum(dv), 1e-30)
        rdv = jnp.ones((L,), jnp.float32) / denom

        def p4(i, c):
            b = i * (L * UNROLL)
            for j in range(UNROLL):
                yqbuf[pl.ds(b + j * L, L)] = yqbuf[pl.ds(b + j * L, L)] * rdv
            return c
        lax.fori_loop(0, NV // UNROLL, p4, 0)

        pltpu.sync_copy(yqbuf, o_hbm.at[row])
        return carry

    lax.fori_loop(0, RPW, do_row, 0)


def kernel(scores, thresholds, y):
    orig_shape = scores.shape
    s2 = scores.reshape(ROWS, COLS)
    thr = jnp.pad(thresholds, (0, 1), constant_values=2.0)
    mesh = plsc.VectorSubcoreMesh(core_axis_name="c", subcore_axis_name="s")
    out = pl.kernel(
        _sc_body,
        out_type=jax.ShapeDtypeStruct((ROWS, COLS), jnp.float32),
        mesh=mesh,
        scratch_types=[
            pltpu.VMEM((COLS,), jnp.float32),
            pltpu.VMEM((COLS,), jnp.float32),
            pltpu.VMEM((L,), jnp.float32),
            pltpu.VMEM((L,), jnp.float32),
            pltpu.VMEM((L,), jnp.float32),
            pltpu.SemaphoreType.DMA,
        ],
        compiler_params=pltpu.CompilerParams(needs_layout_passes=False),
    )(thr, y, s2)
    return out.reshape(orig_shape)
